# baseline (speedup 1.0000x reference)
import dataclasses
import functools

import jax
import jax.numpy as jnp
from jax import lax
from jax.experimental import pallas as pl
from jax.experimental.pallas import tpu as pltpu
from jax.experimental.pallas import tpu_sc as plsc

_SC_CORES = 2
_SC_SUBCORES = 16
_LANES = 16
_CHUNK = 128


def _precompute_tables(latent, w1a, w1b, b1):
    n, d = latent.shape
    h = w1a.shape[1]
    blk = 2000
    dn = (((1,), (0,)), ((), ()))

    def body(lat_ref, w1a_ref, w1b_ref, b1_ref, a_ref, b_ref):
        x = lat_ref[...]
        a_ref[...] = (
            lax.dot_general(x, w1a_ref[...], dn, precision=lax.Precision.HIGHEST)
            + b1_ref[...]
        )
        b_ref[...] = lax.dot_general(
            x, w1b_ref[...], dn, precision=lax.Precision.HIGHEST
        )

    return pl.pallas_call(
        body,
        grid=(n // blk,),
        in_specs=[
            pl.BlockSpec((blk, d), lambda i: (i, 0)),
            pl.BlockSpec((d, h), lambda i: (0, 0)),
            pl.BlockSpec((d, h), lambda i: (0, 0)),
            pl.BlockSpec((1, h), lambda i: (0, 0)),
        ],
        out_specs=[
            pl.BlockSpec((blk, h), lambda i: (i, 0)),
            pl.BlockSpec((blk, h), lambda i: (i, 0)),
        ],
        out_shape=[jax.ShapeDtypeStruct((n, h), jnp.float32)] * 2,
    )(latent, w1a, w1b, b1.reshape(1, h))


_NBUF = 3


def _sc_edge_decode(table_a, table_b, src, dst, w2b2):
    e = src.shape[0]
    h = table_a.shape[1]
    nslc = h // _LANES
    nw = _SC_CORES * _SC_SUBCORES
    n_chunks = e // _CHUNK
    per_worker = -(-n_chunks // nw)

    mesh = plsc.VectorSubcoreMesh(core_axis_name="c", subcore_axis_name="s")

    scratch = (
        [pltpu.VMEM((_CHUNK,), jnp.int32) for _ in range(2 * _NBUF)]
        + [pltpu.VMEM((_CHUNK, h), jnp.float32) for _ in range(2 * _NBUF)]
        + [pltpu.VMEM((_CHUNK,), jnp.float32) for _ in range(_NBUF)]
        + [pltpu.VMEM(w2b2.shape, jnp.float32)]
        + [pltpu.SemaphoreType.DMA for _ in range(2 * _NBUF)]
    )

    cp = pltpu.CompilerParams()
    if "needs_layout_passes" in pltpu.CompilerParams.__dataclass_fields__:
        cp = dataclasses.replace(cp, needs_layout_passes=False)

    @functools.partial(
        pl.kernel,
        mesh=mesh,
        out_type=jax.ShapeDtypeStruct((e,), jnp.float32),
        scratch_types=scratch,
        compiler_params=cp,
    )
    def k(a_hbm, b_hbm, src_hbm, dst_hbm, w_hbm, o_hbm, *bufs):
        idx_s = bufs[0:_NBUF]
        idx_d = bufs[_NBUF:2 * _NBUF]
        buf_a = bufs[2 * _NBUF:3 * _NBUF]
        buf_b = bufs[3 * _NBUF:4 * _NBUF]
        out_v = bufs[4 * _NBUF:5 * _NBUF]
        w_v = bufs[5 * _NBUF]
        sem_g = bufs[5 * _NBUF + 1:6 * _NBUF + 1]
        sem_o = bufs[6 * _NBUF + 1:7 * _NBUF + 1]

        wid = lax.axis_index("s") * _SC_CORES + lax.axis_index("c")
        nv = (n_chunks - 1 - wid) // nw + 1

        pltpu.sync_copy(w_hbm, w_v)
        w2c = [w_v[pl.ds(cc * _LANES, _LANES)] for cc in range(nslc)]
        b2v = w_v[pl.ds(h, _LANES)]

        def prep(c, b):
            base = (c * nw + wid) * _CHUNK
            ci = pltpu.async_copy(src_hbm.at[pl.ds(base, _CHUNK)], idx_s[b],
                                  sem_g[b])
            cj = pltpu.async_copy(dst_hbm.at[pl.ds(base, _CHUNK)], idx_d[b],
                                  sem_g[b])
            ci.wait()
            cj.wait()
            pltpu.async_copy(a_hbm.at[idx_s[b]], buf_a[b], sem_g[b])
            pltpu.async_copy(b_hbm.at[idx_d[b]], buf_b[b], sem_g[b])

        def wait_gathers(b):
            pltpu.make_async_copy(a_hbm.at[idx_s[b]], buf_a[b], sem_g[b]).wait()
            pltpu.make_async_copy(b_hbm.at[idx_d[b]], buf_b[b], sem_g[b]).wait()

        def wait_out(b):
            pltpu.make_async_copy(out_v[b], o_hbm.at[pl.ds(0, _CHUNK)],
                                  sem_o[b]).wait()

        prep(0, 0)

        @pl.loop(0, per_worker, step=_NBUF)
        def _(kk):
            for b in range(_NBUF):
                c = kk + b
                b1 = (b + 1) % _NBUF

                @pl.when(c + 1 < nv)
                def _():
                    prep(c + 1, b1)

                @pl.when(c < nv)
                def _():
                    @pl.when(c >= _NBUF)
                    def _():
                        wait_out(b)

                    wait_gathers(b)

                    lane = lax.iota(jnp.int32, _LANES)

                    @pl.loop(0, _CHUNK, step=_LANES)
                    def _(r0):
                        run = jnp.zeros((_LANES,), jnp.float32)
                        for j in range(_LANES):
                            r = r0 + j
                            acc = None
                            for cc in range(nslc):
                                sl = pl.ds(cc * _LANES, _LANES)
                                t = jnp.maximum(
                                    buf_a[b][r, sl] + buf_b[b][r, sl], 0.0)
                                ft = t * w2c[cc]
                                acc = ft if acc is None else acc + ft
                            run = jnp.where(lane == j, jnp.sum(acc), run)
                        x = run + b2v
                        out_v[b][pl.ds(r0, _LANES)] = 1.0 / (1.0 + jnp.exp(-x))

                    base = (c * nw + wid) * _CHUNK
                    pltpu.async_copy(out_v[b], o_hbm.at[pl.ds(base, _CHUNK)],
                                     sem_o[b])

        for b in range(_NBUF):
            @pl.when(nv > b)
            def _():
                wait_out(b)

    return k(table_a, table_b, src, dst, w2b2)


def kernel(latent_space, edge_index, W1, b1, W2, b2):
    d = latent_space.shape[1]
    src = edge_index[0].astype(jnp.int32)
    dst = edge_index[1].astype(jnp.int32)
    table_a, table_b = _precompute_tables(latent_space, W1[:d], W1[d:], b1)
    w2b2 = jnp.concatenate([W2[:, 0], jnp.full((16,), b2[0], jnp.float32)])
    return _sc_edge_decode(table_a, table_b, src, dst, w2b2)


# speedup vs baseline: 1.0292x; 1.0292x over previous
import dataclasses
import functools

import jax
import jax.numpy as jnp
from jax import lax
from jax.experimental import pallas as pl
from jax.experimental.pallas import tpu as pltpu
from jax.experimental.pallas import tpu_sc as plsc

_SC_CORES = 2
_SC_SUBCORES = 16
_LANES = 16
_CHUNK = 128


def _precompute_tables(latent, w1a, w1b, b1):
    n, d = latent.shape
    h = w1a.shape[1]
    blk = 2000
    dn = (((1,), (0,)), ((), ()))

    def body(lat_ref, w1a_ref, w1b_ref, b1_ref, a_ref, b_ref):
        x = lat_ref[...]
        a_ref[...] = (
            lax.dot_general(x, w1a_ref[...], dn, precision=lax.Precision.HIGHEST)
            + b1_ref[...]
        )
        b_ref[...] = lax.dot_general(
            x, w1b_ref[...], dn, precision=lax.Precision.HIGHEST
        )

    return pl.pallas_call(
        body,
        grid=(n // blk,),
        in_specs=[
            pl.BlockSpec((blk, d), lambda i: (i, 0)),
            pl.BlockSpec((d, h), lambda i: (0, 0)),
            pl.BlockSpec((d, h), lambda i: (0, 0)),
            pl.BlockSpec((1, h), lambda i: (0, 0)),
        ],
        out_specs=[
            pl.BlockSpec((blk, h), lambda i: (i, 0)),
            pl.BlockSpec((blk, h), lambda i: (i, 0)),
        ],
        out_shape=[jax.ShapeDtypeStruct((n, h), jnp.float32)] * 2,
    )(latent, w1a, w1b, b1.reshape(1, h))


_NBUF = 3


def _sc_edge_decode(table_a, table_b, src, dst, w2b2):
    e = src.shape[0]
    h = table_a.shape[1]
    nslc = h // _LANES
    nw = _SC_CORES * _SC_SUBCORES
    n_chunks = e // _CHUNK
    per_worker = -(-n_chunks // nw)

    mesh = plsc.VectorSubcoreMesh(core_axis_name="c", subcore_axis_name="s")

    scratch = (
        [pltpu.VMEM((_CHUNK,), jnp.int32) for _ in range(2 * _NBUF)]
        + [pltpu.VMEM((_CHUNK, h), jnp.float32) for _ in range(2 * _NBUF)]
        + [pltpu.VMEM((_CHUNK,), jnp.float32) for _ in range(_NBUF)]
        + [pltpu.VMEM(w2b2.shape, jnp.float32)]
        + [pltpu.SemaphoreType.DMA for _ in range(2 * _NBUF)]
    )

    cp = pltpu.CompilerParams()
    if "needs_layout_passes" in pltpu.CompilerParams.__dataclass_fields__:
        cp = dataclasses.replace(cp, needs_layout_passes=False)

    @functools.partial(
        pl.kernel,
        mesh=mesh,
        out_type=jax.ShapeDtypeStruct((e,), jnp.float32),
        scratch_types=scratch,
        compiler_params=cp,
    )
    def k(a_hbm, b_hbm, src_hbm, dst_hbm, w_hbm, o_hbm, *bufs):
        idx_s = bufs[0:_NBUF]
        idx_d = bufs[_NBUF:2 * _NBUF]
        buf_a = bufs[2 * _NBUF:3 * _NBUF]
        buf_b = bufs[3 * _NBUF:4 * _NBUF]
        out_v = bufs[4 * _NBUF:5 * _NBUF]
        w_v = bufs[5 * _NBUF]
        sem_g = bufs[5 * _NBUF + 1:6 * _NBUF + 1]
        sem_o = bufs[6 * _NBUF + 1:7 * _NBUF + 1]

        wid = lax.axis_index("s") * _SC_CORES + lax.axis_index("c")
        nv = (n_chunks - 1 - wid) // nw + 1

        pltpu.sync_copy(w_hbm, w_v)
        w2c = [w_v[pl.ds(cc * _LANES, _LANES)] for cc in range(nslc)]
        b2v = w_v[pl.ds(h, _LANES)]

        def prep(c, b):
            base = (c * nw + wid) * _CHUNK
            ci = pltpu.async_copy(src_hbm.at[pl.ds(base, _CHUNK)], idx_s[b],
                                  sem_g[b])
            cj = pltpu.async_copy(dst_hbm.at[pl.ds(base, _CHUNK)], idx_d[b],
                                  sem_g[b])
            ci.wait()
            cj.wait()
            pltpu.async_copy(a_hbm.at[idx_s[b]], buf_a[b], sem_g[b])
            pltpu.async_copy(b_hbm.at[idx_d[b]], buf_b[b], sem_g[b])

        def wait_gathers(b):
            pltpu.make_async_copy(a_hbm.at[idx_s[b]], buf_a[b], sem_g[b]).wait()
            pltpu.make_async_copy(b_hbm.at[idx_d[b]], buf_b[b], sem_g[b]).wait()

        def wait_out(b):
            pltpu.make_async_copy(out_v[b], o_hbm.at[pl.ds(0, _CHUNK)],
                                  sem_o[b]).wait()

        prep(0, 0)

        @pl.loop(0, per_worker, step=_NBUF)
        def _(kk):
            for b in range(_NBUF):
                c = kk + b
                b1 = (b + 1) % _NBUF

                @pl.when(c + 1 < nv)
                def _():
                    prep(c + 1, b1)

                @pl.when(c < nv)
                def _():
                    @pl.when(c >= _NBUF)
                    def _():
                        wait_out(b)

                    wait_gathers(b)

                    lane = lax.iota(jnp.int32, _LANES)
                    rot = {
                        k: (lane + k) % _LANES
                        for k in (8, 4, 12, 2, 14, 1, 15)
                    }
                    masks = {
                        g: (lane % g) < (g // 2) for g in (16, 8, 4, 2)
                    }
                    pib = "wrap"

                    def merge(u, v, g):
                        k = g // 2
                        ul = jnp.take(u, rot[k], mode=pib)
                        vr = jnp.take(v, rot[_LANES - k], mode=pib)
                        return jnp.where(masks[g], u + ul, v + vr)

                    brev = [0, 8, 4, 12, 2, 10, 6, 14,
                            1, 9, 5, 13, 3, 11, 7, 15]

                    @pl.loop(0, _CHUNK, step=_LANES)
                    def _(r0):
                        def edge_acc(j):
                            r = r0 + brev[j]
                            acc = None
                            for cc in range(nslc):
                                sl = pl.ds(cc * _LANES, _LANES)
                                t = jnp.maximum(
                                    buf_a[b][r, sl] + buf_b[b][r, sl], 0.0)
                                ft = t * w2c[cc]
                                acc = ft if acc is None else acc + ft
                            return acc

                        def build(lo, size):
                            if size == 2:
                                return merge(edge_acc(lo), edge_acc(lo + 1), 16)
                            half = size // 2
                            return merge(build(lo, half), build(lo + half, half),
                                         32 // size)

                        x = build(0, _LANES) + b2v
                        out_v[b][pl.ds(r0, _LANES)] = 1.0 / (1.0 + jnp.exp(-x))

                    base = (c * nw + wid) * _CHUNK
                    pltpu.async_copy(out_v[b], o_hbm.at[pl.ds(base, _CHUNK)],
                                     sem_o[b])

        for b in range(_NBUF):
            @pl.when(nv > b)
            def _():
                wait_out(b)

    return k(table_a, table_b, src, dst, w2b2)


def kernel(latent_space, edge_index, W1, b1, W2, b2):
    d = latent_space.shape[1]
    src = edge_index[0].astype(jnp.int32)
    dst = edge_index[1].astype(jnp.int32)
    table_a, table_b = _precompute_tables(latent_space, W1[:d], W1[d:], b1)
    w2b2 = jnp.concatenate([W2[:, 0], jnp.full((16,), b2[0], jnp.float32)])
    return _sc_edge_decode(table_a, table_b, src, dst, w2b2)


# speedup vs baseline: 1.1305x; 1.0984x over previous
import dataclasses
import functools

import jax
import jax.numpy as jnp
from jax import lax
from jax.experimental import pallas as pl
from jax.experimental.pallas import tpu as pltpu
from jax.experimental.pallas import tpu_sc as plsc

_SC_CORES = 2
_SC_SUBCORES = 16
_LANES = 16
_CHUNK = 128


def _precompute_tables(latent, w1a, w1b, b1):
    n, d = latent.shape
    h = w1a.shape[1]
    blk = 2000
    dn = (((1,), (0,)), ((), ()))

    def body(lat_ref, w1a_ref, w1b_ref, b1_ref, a_ref, b_ref):
        x = lat_ref[...]
        a_ref[...] = (
            lax.dot_general(x, w1a_ref[...], dn, precision=lax.Precision.HIGHEST)
            + b1_ref[...]
        )
        b_ref[...] = lax.dot_general(
            x, w1b_ref[...], dn, precision=lax.Precision.HIGHEST
        )

    return pl.pallas_call(
        body,
        grid=(n // blk,),
        in_specs=[
            pl.BlockSpec((blk, d), lambda i: (i, 0)),
            pl.BlockSpec((d, h), lambda i: (0, 0)),
            pl.BlockSpec((d, h), lambda i: (0, 0)),
            pl.BlockSpec((1, h), lambda i: (0, 0)),
        ],
        out_specs=[
            pl.BlockSpec((blk, h), lambda i: (i, 0)),
            pl.BlockSpec((blk, h), lambda i: (i, 0)),
        ],
        out_shape=[jax.ShapeDtypeStruct((n, h), jnp.float32)] * 2,
    )(latent, w1a, w1b, b1.reshape(1, h))


_NBUF = 3


def _sc_edge_decode(table_a, table_b, src, dst, w2b2):
    e = src.shape[0]
    h = table_a.shape[1]
    nslc = h // _LANES
    nw = _SC_CORES * _SC_SUBCORES
    n_chunks = e // _CHUNK
    per_worker = -(-n_chunks // nw)

    mesh = plsc.VectorSubcoreMesh(core_axis_name="c", subcore_axis_name="s")

    scratch = (
        [pltpu.VMEM((_CHUNK,), jnp.int32) for _ in range(2 * _NBUF)]
        + [pltpu.VMEM((_CHUNK, h), jnp.float32) for _ in range(2 * _NBUF)]
        + [pltpu.VMEM((_CHUNK,), jnp.float32) for _ in range(_NBUF)]
        + [pltpu.VMEM(w2b2.shape, jnp.float32)]
        + [pltpu.SemaphoreType.DMA for _ in range(3 * _NBUF)]
    )

    cp = pltpu.CompilerParams()
    if "needs_layout_passes" in pltpu.CompilerParams.__dataclass_fields__:
        cp = dataclasses.replace(cp, needs_layout_passes=False)

    @functools.partial(
        pl.kernel,
        mesh=mesh,
        out_type=jax.ShapeDtypeStruct((e,), jnp.float32),
        scratch_types=scratch,
        compiler_params=cp,
    )
    def k(a_hbm, b_hbm, src_hbm, dst_hbm, w_hbm, o_hbm, *bufs):
        idx_s = bufs[0:_NBUF]
        idx_d = bufs[_NBUF:2 * _NBUF]
        buf_a = bufs[2 * _NBUF:3 * _NBUF]
        buf_b = bufs[3 * _NBUF:4 * _NBUF]
        out_v = bufs[4 * _NBUF:5 * _NBUF]
        w_v = bufs[5 * _NBUF]
        sem_g = bufs[5 * _NBUF + 1:6 * _NBUF + 1]
        sem_o = bufs[6 * _NBUF + 1:7 * _NBUF + 1]
        sem_i = bufs[7 * _NBUF + 1:8 * _NBUF + 1]

        wid = lax.axis_index("s") * _SC_CORES + lax.axis_index("c")
        nv = (n_chunks - 1 - wid) // nw + 1

        pltpu.sync_copy(w_hbm, w_v)
        w2c = [w_v[pl.ds(cc * _LANES, _LANES)] for cc in range(nslc)]
        b2v = w_v[pl.ds(h, _LANES)]

        def fetch_idx(c, b):
            base = (c * nw + wid) * _CHUNK
            pltpu.async_copy(src_hbm.at[pl.ds(base, _CHUNK)], idx_s[b],
                             sem_i[b])
            pltpu.async_copy(dst_hbm.at[pl.ds(base, _CHUNK)], idx_d[b],
                             sem_i[b])

        def start_gathers(b):
            pltpu.make_async_copy(src_hbm.at[pl.ds(0, _CHUNK)], idx_s[b],
                                  sem_i[b]).wait()
            pltpu.make_async_copy(dst_hbm.at[pl.ds(0, _CHUNK)], idx_d[b],
                                  sem_i[b]).wait()
            pltpu.async_copy(a_hbm.at[idx_s[b]], buf_a[b], sem_g[b])
            pltpu.async_copy(b_hbm.at[idx_d[b]], buf_b[b], sem_g[b])

        def wait_gathers(b):
            pltpu.make_async_copy(a_hbm.at[idx_s[b]], buf_a[b], sem_g[b]).wait()
            pltpu.make_async_copy(b_hbm.at[idx_d[b]], buf_b[b], sem_g[b]).wait()

        def wait_out(b):
            pltpu.make_async_copy(out_v[b], o_hbm.at[pl.ds(0, _CHUNK)],
                                  sem_o[b]).wait()

        for c0 in range(_NBUF):
            @pl.when(c0 < nv)
            def _():
                fetch_idx(c0, c0)
        for c0 in range(2):
            @pl.when(c0 < nv)
            def _():
                start_gathers(c0)

        @pl.loop(0, per_worker, step=_NBUF)
        def _(kk):
            for b in range(_NBUF):
                c = kk + b
                b2 = (b + 2) % _NBUF

                @pl.when(c < nv)
                def _():
                    @pl.when(c >= _NBUF)
                    def _():
                        wait_out(b)

                    wait_gathers(b)

                    @pl.when(c + _NBUF < nv)
                    def _():
                        fetch_idx(c + _NBUF, b)

                    lane = lax.iota(jnp.int32, _LANES)
                    rot = {
                        k: (lane + k) % _LANES
                        for k in (8, 4, 12, 2, 14, 1, 15)
                    }
                    masks = {
                        g: (lane % g) < (g // 2) for g in (16, 8, 4, 2)
                    }
                    pib = "wrap"

                    def merge(u, v, g):
                        k = g // 2
                        ul = jnp.take(u, rot[k], mode=pib)
                        vr = jnp.take(v, rot[_LANES - k], mode=pib)
                        return jnp.where(masks[g], u + ul, v + vr)

                    brev = [0, 8, 4, 12, 2, 10, 6, 14,
                            1, 9, 5, 13, 3, 11, 7, 15]

                    @pl.loop(0, _CHUNK, step=_LANES)
                    def _(r0):
                        def edge_acc(j):
                            r = r0 + brev[j]
                            acc = None
                            for cc in range(nslc):
                                sl = pl.ds(cc * _LANES, _LANES)
                                t = jnp.maximum(
                                    buf_a[b][r, sl] + buf_b[b][r, sl], 0.0)
                                ft = t * w2c[cc]
                                acc = ft if acc is None else acc + ft
                            return acc

                        def build(lo, size):
                            if size == 2:
                                return merge(edge_acc(lo), edge_acc(lo + 1), 16)
                            half = size // 2
                            return merge(build(lo, half), build(lo + half, half),
                                         32 // size)

                        x = build(0, _LANES) + b2v
                        out_v[b][pl.ds(r0, _LANES)] = 1.0 / (1.0 + jnp.exp(-x))

                    base = (c * nw + wid) * _CHUNK
                    pltpu.async_copy(out_v[b], o_hbm.at[pl.ds(base, _CHUNK)],
                                     sem_o[b])

                    @pl.when(c + 2 < nv)
                    def _():
                        start_gathers(b2)

        for b in range(_NBUF):
            @pl.when(nv > b)
            def _():
                wait_out(b)

    return k(table_a, table_b, src, dst, w2b2)


def kernel(latent_space, edge_index, W1, b1, W2, b2):
    d = latent_space.shape[1]
    src = edge_index[0].astype(jnp.int32)
    dst = edge_index[1].astype(jnp.int32)
    table_a, table_b = _precompute_tables(latent_space, W1[:d], W1[d:], b1)
    w2b2 = jnp.concatenate([W2[:, 0], jnp.full((16,), b2[0], jnp.float32)])
    return _sc_edge_decode(table_a, table_b, src, dst, w2b2)


# speedup vs baseline: 1.7372x; 1.5367x over previous
import dataclasses
import functools

import jax
import jax.numpy as jnp
from jax import lax
from jax.experimental import pallas as pl
from jax.experimental.pallas import tpu as pltpu
from jax.experimental.pallas import tpu_sc as plsc

_SC_CORES = 2
_SC_SUBCORES = 16
_LANES = 16
_CHUNK = 128


def _precompute_tables(latent, w1a, w1b, b1):
    n, d = latent.shape
    h = w1a.shape[1]
    blk = 2000
    dn = (((1,), (0,)), ((), ()))

    def body(lat_ref, w1a_ref, w1b_ref, b1_ref, a_ref, b_ref):
        x = lat_ref[...]
        a_ref[...] = (
            lax.dot_general(x, w1a_ref[...], dn, precision=lax.Precision.HIGHEST)
            + b1_ref[...]
        ).astype(jnp.bfloat16)
        b_ref[...] = lax.dot_general(
            x, w1b_ref[...], dn, precision=lax.Precision.HIGHEST
        ).astype(jnp.bfloat16)

    return pl.pallas_call(
        body,
        grid=(n // blk,),
        in_specs=[
            pl.BlockSpec((blk, d), lambda i: (i, 0)),
            pl.BlockSpec((d, h), lambda i: (0, 0)),
            pl.BlockSpec((d, h), lambda i: (0, 0)),
            pl.BlockSpec((1, h), lambda i: (0, 0)),
        ],
        out_specs=[
            pl.BlockSpec((blk, h), lambda i: (i, 0)),
            pl.BlockSpec((blk, h), lambda i: (i, 0)),
        ],
        out_shape=[jax.ShapeDtypeStruct((n, h), jnp.bfloat16)] * 2,
    )(latent, w1a, w1b, b1.reshape(1, h))


_NBUF = 3


def _sc_edge_decode(table_p, src, dst, w2bf, b2rep):
    e = src.shape[0]
    h = table_p.shape[1]
    half = h // 2
    nslc2 = half // _LANES
    nw = _SC_CORES * _SC_SUBCORES
    n_chunks = e // _CHUNK
    per_worker = -(-n_chunks // nw)

    mesh = plsc.VectorSubcoreMesh(core_axis_name="c", subcore_axis_name="s")

    scratch = (
        [pltpu.VMEM((_CHUNK,), jnp.int32) for _ in range(2 * _NBUF)]
        + [pltpu.VMEM((_CHUNK, h), jnp.float32) for _ in range(2 * _NBUF)]
        + [pltpu.VMEM((_CHUNK,), jnp.float32) for _ in range(_NBUF)]
        + [pltpu.VMEM(w2bf.shape, jnp.float32),
           pltpu.VMEM(b2rep.shape, jnp.float32)]
        + [pltpu.SemaphoreType.DMA for _ in range(3 * _NBUF)]
    )

    cp = pltpu.CompilerParams()
    if "needs_layout_passes" in pltpu.CompilerParams.__dataclass_fields__:
        cp = dataclasses.replace(cp, needs_layout_passes=False)

    @functools.partial(
        pl.kernel,
        mesh=mesh,
        out_type=jax.ShapeDtypeStruct((e,), jnp.float32),
        scratch_types=scratch,
        compiler_params=cp,
    )
    def k(p_hbm, src_hbm, dst_hbm, w_hbm, b2_hbm, o_hbm, *bufs):
        idx_s = bufs[0:_NBUF]
        idx_d = bufs[_NBUF:2 * _NBUF]
        buf_a = bufs[2 * _NBUF:3 * _NBUF]
        buf_b = bufs[3 * _NBUF:4 * _NBUF]
        out_v = bufs[4 * _NBUF:5 * _NBUF]
        w_v = bufs[5 * _NBUF]
        b2_v = bufs[5 * _NBUF + 1]
        sem_g = bufs[5 * _NBUF + 2:6 * _NBUF + 2]
        sem_o = bufs[6 * _NBUF + 2:7 * _NBUF + 2]
        sem_i = bufs[7 * _NBUF + 2:8 * _NBUF + 2]

        wid = lax.axis_index("s") * _SC_CORES + lax.axis_index("c")
        nv = (n_chunks - 1 - wid) // nw + 1

        pltpu.sync_copy(w_hbm, w_v)
        pltpu.sync_copy(b2_hbm, b2_v)
        w2c = [plsc.bitcast(w_v[pl.ds(cc * _LANES, _LANES)], jnp.bfloat16)
               for cc in range(nslc2)]
        b2v = b2_v[pl.ds(0, _LANES)]

        def fetch_idx(c, b):
            base = (c * nw + wid) * _CHUNK
            pltpu.async_copy(src_hbm.at[pl.ds(base, _CHUNK)], idx_s[b],
                             sem_i[b])
            pltpu.async_copy(dst_hbm.at[pl.ds(base, _CHUNK)], idx_d[b],
                             sem_i[b])

        def start_gathers(b):
            pltpu.make_async_copy(src_hbm.at[pl.ds(0, _CHUNK)], idx_s[b],
                                  sem_i[b]).wait()
            pltpu.make_async_copy(dst_hbm.at[pl.ds(0, _CHUNK)], idx_d[b],
                                  sem_i[b]).wait()
            pltpu.async_copy(p_hbm.at[idx_s[b]], buf_a[b], sem_g[b])
            pltpu.async_copy(p_hbm.at[idx_d[b]], buf_b[b], sem_g[b])

        def wait_gathers(b):
            pltpu.make_async_copy(p_hbm.at[idx_s[b]], buf_a[b], sem_g[b]).wait()
            pltpu.make_async_copy(p_hbm.at[idx_d[b]], buf_b[b], sem_g[b]).wait()

        def wait_out(b):
            pltpu.make_async_copy(out_v[b], o_hbm.at[pl.ds(0, _CHUNK)],
                                  sem_o[b]).wait()

        for c0 in range(_NBUF):
            @pl.when(c0 < nv)
            def _():
                fetch_idx(c0, c0)
        for c0 in range(2):
            @pl.when(c0 < nv)
            def _():
                start_gathers(c0)

        @pl.loop(0, per_worker, step=_NBUF)
        def _(kk):
            for b in range(_NBUF):
                c = kk + b
                b2 = (b + 2) % _NBUF

                @pl.when(c < nv)
                def _():
                    @pl.when(c >= _NBUF)
                    def _():
                        wait_out(b)

                    wait_gathers(b)

                    @pl.when(c + _NBUF < nv)
                    def _():
                        fetch_idx(c + _NBUF, b)

                    lane = lax.iota(jnp.int32, _LANES)
                    rot = {
                        k: (lane + k) % _LANES
                        for k in (8, 4, 12, 2, 14, 1, 15)
                    }
                    masks = {
                        g: (lane % g) < (g // 2) for g in (16, 8, 4, 2)
                    }
                    pib = "wrap"

                    def merge(u, v, g):
                        k = g // 2
                        ul = jnp.take(u, rot[k], mode=pib)
                        vr = jnp.take(v, rot[_LANES - k], mode=pib)
                        return jnp.where(masks[g], u + ul, v + vr)

                    brev = [0, 8, 4, 12, 2, 10, 6, 14,
                            1, 9, 5, 13, 3, 11, 7, 15]

                    @pl.loop(0, _CHUNK, step=_LANES)
                    def _(r0):
                        def edge_acc(j):
                            r = r0 + brev[j]
                            acc = None
                            for cc in range(nslc2):
                                sla = pl.ds(cc * _LANES, _LANES)
                                slb = pl.ds(half + cc * _LANES, _LANES)
                                av = plsc.bitcast(buf_a[b][r, sla], jnp.bfloat16)
                                bv = plsc.bitcast(buf_b[b][r, slb], jnp.bfloat16)
                                t = jnp.maximum(av + bv, 0.0)
                                p0, p1 = plsc.unpack(
                                    t * w2c[cc],
                                    format=plsc.PackFormat.INTERLEAVED)
                                ps = p0 + p1
                                acc = ps if acc is None else acc + ps
                            return acc

                        def build(lo, size):
                            if size == 2:
                                return merge(edge_acc(lo), edge_acc(lo + 1), 16)
                            half = size // 2
                            return merge(build(lo, half), build(lo + half, half),
                                         32 // size)

                        x = build(0, _LANES) + b2v
                        out_v[b][pl.ds(r0, _LANES)] = 1.0 / (1.0 + jnp.exp(-x))

                    base = (c * nw + wid) * _CHUNK
                    pltpu.async_copy(out_v[b], o_hbm.at[pl.ds(base, _CHUNK)],
                                     sem_o[b])

                    @pl.when(c + 2 < nv)
                    def _():
                        start_gathers(b2)

        for b in range(_NBUF):
            @pl.when(nv > b)
            def _():
                wait_out(b)

    return k(table_p, src, dst, w2bf, b2rep)


def kernel(latent_space, edge_index, W1, b1, W2, b2):
    d = latent_space.shape[1]
    src = edge_index[0].astype(jnp.int32)
    dst = edge_index[1].astype(jnp.int32)
    table_a, table_b = _precompute_tables(latent_space, W1[:d], W1[d:], b1)

    def pack_words(x):
        return lax.bitcast_convert_type(
            x.reshape(*x.shape[:-1], x.shape[-1] // 2, 2), jnp.float32)

    w2bf = pack_words(W2[:, 0].astype(jnp.bfloat16))
    b2rep = jnp.full((16,), b2[0], jnp.float32)
    table_p = jnp.concatenate([pack_words(table_a), pack_words(table_b)],
                              axis=1)
    return _sc_edge_decode(table_p, src, dst, w2bf, b2rep)


# speedup vs baseline: 2.2988x; 1.3232x over previous
import dataclasses
import functools

import jax
import jax.numpy as jnp
from jax import lax
from jax.experimental import pallas as pl
from jax.experimental.pallas import tpu as pltpu
from jax.experimental.pallas import tpu_sc as plsc

_SC_CORES = 2
_SC_SUBCORES = 16
_LANES = 16
_CHUNK = 128


def _pack_halves(v):
    k = v.shape[-1] // 2
    lo = v[..., :k].astype(jnp.bfloat16)
    hi = v[..., k:].astype(jnp.bfloat16)
    u1 = lax.bitcast_convert_type(lo, jnp.uint16).astype(jnp.uint32)
    u2 = lax.bitcast_convert_type(hi, jnp.uint16).astype(jnp.uint32)
    return lax.bitcast_convert_type(u1 | (u2 << 16), jnp.float32)


def _precompute_table(latent, w1a, w1b, b1):
    n, d = latent.shape
    h = w1a.shape[1]
    blk = 2000
    dn = (((1,), (0,)), ((), ()))

    def body(lat_ref, w1a_ref, w1b_ref, b1_ref, o_ref):
        x = lat_ref[...]
        a = lax.dot_general(x, w1a_ref[...], dn,
                            precision=lax.Precision.HIGHEST) + b1_ref[...]
        bt = lax.dot_general(x, w1b_ref[...], dn,
                             precision=lax.Precision.HIGHEST)
        o_ref[:, :h // 2] = _pack_halves(a)
        o_ref[:, h // 2:] = _pack_halves(bt)

    return pl.pallas_call(
        body,
        grid=(n // blk,),
        in_specs=[
            pl.BlockSpec((blk, d), lambda i: (i, 0)),
            pl.BlockSpec((d, h), lambda i: (0, 0)),
            pl.BlockSpec((d, h), lambda i: (0, 0)),
            pl.BlockSpec((1, h), lambda i: (0, 0)),
        ],
        out_specs=pl.BlockSpec((blk, h), lambda i: (i, 0)),
        out_shape=jax.ShapeDtypeStruct((n, h), jnp.float32),
    )(latent, w1a, w1b, b1.reshape(1, h))


_NBUF = 3


def _sc_edge_decode(table_p, src, dst, w2bf, b2rep):
    e = src.shape[0]
    h = table_p.shape[1]
    half = h // 2
    nslc2 = half // _LANES
    nw = _SC_CORES * _SC_SUBCORES
    n_chunks = e // _CHUNK
    per_worker = -(-n_chunks // nw)

    mesh = plsc.VectorSubcoreMesh(core_axis_name="c", subcore_axis_name="s")

    scratch = (
        [pltpu.VMEM((_CHUNK,), jnp.int32) for _ in range(2 * _NBUF)]
        + [pltpu.VMEM((_CHUNK, h), jnp.float32) for _ in range(2 * _NBUF)]
        + [pltpu.VMEM((_CHUNK,), jnp.float32) for _ in range(_NBUF)]
        + [pltpu.VMEM(w2bf.shape, jnp.float32),
           pltpu.VMEM(b2rep.shape, jnp.float32)]
        + [pltpu.SemaphoreType.DMA for _ in range(3 * _NBUF)]
    )

    cp = pltpu.CompilerParams()
    if "needs_layout_passes" in pltpu.CompilerParams.__dataclass_fields__:
        cp = dataclasses.replace(cp, needs_layout_passes=False)

    @functools.partial(
        pl.kernel,
        mesh=mesh,
        out_type=jax.ShapeDtypeStruct((e,), jnp.float32),
        scratch_types=scratch,
        compiler_params=cp,
    )
    def k(p_hbm, src_hbm, dst_hbm, w_hbm, b2_hbm, o_hbm, *bufs):
        idx_s = bufs[0:_NBUF]
        idx_d = bufs[_NBUF:2 * _NBUF]
        buf_a = bufs[2 * _NBUF:3 * _NBUF]
        buf_b = bufs[3 * _NBUF:4 * _NBUF]
        out_v = bufs[4 * _NBUF:5 * _NBUF]
        w_v = bufs[5 * _NBUF]
        b2_v = bufs[5 * _NBUF + 1]
        sem_g = bufs[5 * _NBUF + 2:6 * _NBUF + 2]
        sem_o = bufs[6 * _NBUF + 2:7 * _NBUF + 2]
        sem_i = bufs[7 * _NBUF + 2:8 * _NBUF + 2]

        wid = lax.axis_index("s") * _SC_CORES + lax.axis_index("c")
        nv = (n_chunks - 1 - wid) // nw + 1

        pltpu.sync_copy(w_hbm, w_v)
        pltpu.sync_copy(b2_hbm, b2_v)
        w2c = [plsc.bitcast(w_v[pl.ds(cc * _LANES, _LANES)], jnp.bfloat16)
               for cc in range(nslc2)]
        b2v = b2_v[pl.ds(0, _LANES)]

        def fetch_idx(c, b):
            base = (c * nw + wid) * _CHUNK
            pltpu.async_copy(src_hbm.at[pl.ds(base, _CHUNK)], idx_s[b],
                             sem_i[b])
            pltpu.async_copy(dst_hbm.at[pl.ds(base, _CHUNK)], idx_d[b],
                             sem_i[b])

        def start_gathers(b):
            pltpu.make_async_copy(src_hbm.at[pl.ds(0, _CHUNK)], idx_s[b],
                                  sem_i[b]).wait()
            pltpu.make_async_copy(dst_hbm.at[pl.ds(0, _CHUNK)], idx_d[b],
                                  sem_i[b]).wait()
            pltpu.async_copy(p_hbm.at[idx_s[b]], buf_a[b], sem_g[b])
            pltpu.async_copy(p_hbm.at[idx_d[b]], buf_b[b], sem_g[b])

        def wait_gathers(b):
            pltpu.make_async_copy(p_hbm.at[idx_s[b]], buf_a[b], sem_g[b]).wait()
            pltpu.make_async_copy(p_hbm.at[idx_d[b]], buf_b[b], sem_g[b]).wait()

        def wait_out(b):
            pltpu.make_async_copy(out_v[b], o_hbm.at[pl.ds(0, _CHUNK)],
                                  sem_o[b]).wait()

        for c0 in range(_NBUF):
            @pl.when(c0 < nv)
            def _():
                fetch_idx(c0, c0)
        for c0 in range(2):
            @pl.when(c0 < nv)
            def _():
                start_gathers(c0)

        @pl.loop(0, per_worker, step=_NBUF)
        def _(kk):
            for b in range(_NBUF):
                c = kk + b
                b2 = (b + 2) % _NBUF

                @pl.when(c < nv)
                def _():
                    @pl.when(c >= _NBUF)
                    def _():
                        wait_out(b)

                    wait_gathers(b)

                    @pl.when(c + _NBUF < nv)
                    def _():
                        fetch_idx(c + _NBUF, b)

                    lane = lax.iota(jnp.int32, _LANES)
                    rot = {
                        k: (lane + k) % _LANES
                        for k in (8, 4, 12, 2, 14, 1, 15)
                    }
                    masks = {
                        g: (lane % g) < (g // 2) for g in (16, 8, 4, 2)
                    }
                    pib = "wrap"

                    def merge(u, v, g):
                        k = g // 2
                        ul = jnp.take(u, rot[k], mode=pib)
                        vr = jnp.take(v, rot[_LANES - k], mode=pib)
                        return jnp.where(masks[g], u + ul, v + vr)

                    brev = [0, 8, 4, 12, 2, 10, 6, 14,
                            1, 9, 5, 13, 3, 11, 7, 15]

                    @pl.loop(0, _CHUNK, step=_LANES)
                    def _(r0):
                        def edge_acc(j):
                            r = r0 + brev[j]
                            acc = None
                            for cc in range(nslc2):
                                sla = pl.ds(cc * _LANES, _LANES)
                                slb = pl.ds(half + cc * _LANES, _LANES)
                                av = plsc.bitcast(buf_a[b][r, sla], jnp.bfloat16)
                                bv = plsc.bitcast(buf_b[b][r, slb], jnp.bfloat16)
                                t = jnp.maximum(av + bv, 0.0)
                                p0, p1 = plsc.unpack(
                                    t * w2c[cc],
                                    format=plsc.PackFormat.INTERLEAVED)
                                ps = p0 + p1
                                acc = ps if acc is None else acc + ps
                            return acc

                        def build(lo, size):
                            if size == 2:
                                return merge(edge_acc(lo), edge_acc(lo + 1), 16)
                            half = size // 2
                            return merge(build(lo, half), build(lo + half, half),
                                         32 // size)

                        x = build(0, _LANES) + b2v
                        out_v[b][pl.ds(r0, _LANES)] = 1.0 / (1.0 + jnp.exp(-x))

                    base = (c * nw + wid) * _CHUNK
                    pltpu.async_copy(out_v[b], o_hbm.at[pl.ds(base, _CHUNK)],
                                     sem_o[b])

                    @pl.when(c + 2 < nv)
                    def _():
                        start_gathers(b2)

        for b in range(_NBUF):
            @pl.when(nv > b)
            def _():
                wait_out(b)

    return k(table_p, src, dst, w2bf, b2rep)


def kernel(latent_space, edge_index, W1, b1, W2, b2):
    d = latent_space.shape[1]
    src = edge_index[0].astype(jnp.int32)
    dst = edge_index[1].astype(jnp.int32)
    table_p = _precompute_table(latent_space, W1[:d], W1[d:], b1)
    w2bf = _pack_halves(W2[:, 0])
    b2rep = jnp.full((16,), b2[0], jnp.float32)
    return _sc_edge_decode(table_p, src, dst, w2bf, b2rep)


# speedup vs baseline: 2.3740x; 1.0327x over previous
import dataclasses
import functools

import jax
import jax.numpy as jnp
from jax import lax
from jax.experimental import pallas as pl
from jax.experimental.pallas import tpu as pltpu
from jax.experimental.pallas import tpu_sc as plsc

_SC_CORES = 2
_SC_SUBCORES = 16
_LANES = 16
_CHUNK = 128


def _pack_halves(v):
    k = v.shape[-1] // 2
    lo = v[..., :k].astype(jnp.bfloat16)
    hi = v[..., k:].astype(jnp.bfloat16)
    u1 = lax.bitcast_convert_type(lo, jnp.uint16).astype(jnp.uint32)
    u2 = lax.bitcast_convert_type(hi, jnp.uint16).astype(jnp.uint32)
    return lax.bitcast_convert_type(u1 | (u2 << 16), jnp.float32)


def _precompute_table(latent, w1a, w1b, b1):
    n, d = latent.shape
    h = w1a.shape[1]
    blk = 2000
    dn = (((1,), (0,)), ((), ()))

    def body(lat_ref, w1a_ref, w1b_ref, b1_ref, o_ref):
        x = lat_ref[...]
        a = lax.dot_general(x, w1a_ref[...], dn,
                            precision=lax.Precision.DEFAULT) + b1_ref[...]
        bt = lax.dot_general(x, w1b_ref[...], dn,
                             precision=lax.Precision.DEFAULT)
        o_ref[:, :h // 2] = _pack_halves(a)
        o_ref[:, h // 2:] = _pack_halves(bt)

    return pl.pallas_call(
        body,
        grid=(n // blk,),
        in_specs=[
            pl.BlockSpec((blk, d), lambda i: (i, 0)),
            pl.BlockSpec((d, h), lambda i: (0, 0)),
            pl.BlockSpec((d, h), lambda i: (0, 0)),
            pl.BlockSpec((1, h), lambda i: (0, 0)),
        ],
        out_specs=pl.BlockSpec((blk, h), lambda i: (i, 0)),
        out_shape=jax.ShapeDtypeStruct((n, h), jnp.float32),
    )(latent, w1a, w1b, b1.reshape(1, h))


_NBUF = 3


def _sc_edge_decode(table_p, src, dst, w2bf, b2rep):
    e = src.shape[0]
    h = table_p.shape[1]
    half = h // 2
    nslc2 = half // _LANES
    nw = _SC_CORES * _SC_SUBCORES
    n_chunks = e // _CHUNK
    per_worker = -(-n_chunks // nw)

    mesh = plsc.VectorSubcoreMesh(core_axis_name="c", subcore_axis_name="s")

    scratch = (
        [pltpu.VMEM((_CHUNK,), jnp.int32) for _ in range(2 * _NBUF)]
        + [pltpu.VMEM((_CHUNK, h), jnp.float32) for _ in range(2 * _NBUF)]
        + [pltpu.VMEM((_CHUNK,), jnp.float32) for _ in range(_NBUF)]
        + [pltpu.VMEM(w2bf.shape, jnp.float32),
           pltpu.VMEM(b2rep.shape, jnp.float32)]
        + [pltpu.SemaphoreType.DMA for _ in range(3 * _NBUF)]
    )

    cp = pltpu.CompilerParams()
    if "needs_layout_passes" in pltpu.CompilerParams.__dataclass_fields__:
        cp = dataclasses.replace(cp, needs_layout_passes=False)

    @functools.partial(
        pl.kernel,
        mesh=mesh,
        out_type=jax.ShapeDtypeStruct((e,), jnp.float32),
        scratch_types=scratch,
        compiler_params=cp,
    )
    def k(p_hbm, src_hbm, dst_hbm, w_hbm, b2_hbm, o_hbm, *bufs):
        idx_s = bufs[0:_NBUF]
        idx_d = bufs[_NBUF:2 * _NBUF]
        buf_a = bufs[2 * _NBUF:3 * _NBUF]
        buf_b = bufs[3 * _NBUF:4 * _NBUF]
        out_v = bufs[4 * _NBUF:5 * _NBUF]
        w_v = bufs[5 * _NBUF]
        b2_v = bufs[5 * _NBUF + 1]
        sem_g = bufs[5 * _NBUF + 2:6 * _NBUF + 2]
        sem_o = bufs[6 * _NBUF + 2:7 * _NBUF + 2]
        sem_i = bufs[7 * _NBUF + 2:8 * _NBUF + 2]

        wid = lax.axis_index("s") * _SC_CORES + lax.axis_index("c")
        nv = (n_chunks - 1 - wid) // nw + 1

        pltpu.sync_copy(w_hbm, w_v)
        pltpu.sync_copy(b2_hbm, b2_v)
        w2c = [plsc.bitcast(w_v[pl.ds(cc * _LANES, _LANES)], jnp.bfloat16)
               for cc in range(nslc2)]
        b2v = b2_v[pl.ds(0, _LANES)]

        def fetch_idx(c, b):
            base = (c * nw + wid) * _CHUNK
            pltpu.async_copy(src_hbm.at[pl.ds(base, _CHUNK)], idx_s[b],
                             sem_i[b])
            pltpu.async_copy(dst_hbm.at[pl.ds(base, _CHUNK)], idx_d[b],
                             sem_i[b])

        def start_gathers(b):
            pltpu.make_async_copy(src_hbm.at[pl.ds(0, _CHUNK)], idx_s[b],
                                  sem_i[b]).wait()
            pltpu.make_async_copy(dst_hbm.at[pl.ds(0, _CHUNK)], idx_d[b],
                                  sem_i[b]).wait()
            pltpu.async_copy(p_hbm.at[idx_s[b]], buf_a[b], sem_g[b])
            pltpu.async_copy(p_hbm.at[idx_d[b]], buf_b[b], sem_g[b])

        def wait_gathers(b):
            pltpu.make_async_copy(p_hbm.at[idx_s[b]], buf_a[b], sem_g[b]).wait()
            pltpu.make_async_copy(p_hbm.at[idx_d[b]], buf_b[b], sem_g[b]).wait()

        def wait_out(b):
            pltpu.make_async_copy(out_v[b], o_hbm.at[pl.ds(0, _CHUNK)],
                                  sem_o[b]).wait()

        for c0 in range(_NBUF):
            @pl.when(c0 < nv)
            def _():
                fetch_idx(c0, c0)
        for c0 in range(2):
            @pl.when(c0 < nv)
            def _():
                start_gathers(c0)

        @pl.loop(0, per_worker, step=_NBUF)
        def _(kk):
            for b in range(_NBUF):
                c = kk + b
                b2 = (b + 2) % _NBUF

                @pl.when(c < nv)
                def _():
                    @pl.when(c >= _NBUF)
                    def _():
                        wait_out(b)

                    wait_gathers(b)

                    @pl.when(c + _NBUF < nv)
                    def _():
                        fetch_idx(c + _NBUF, b)

                    lane = lax.iota(jnp.int32, _LANES)
                    rot = {
                        k: (lane + k) % _LANES
                        for k in (8, 4, 12, 2, 14, 1, 15)
                    }
                    masks = {
                        g: (lane % g) < (g // 2) for g in (16, 8, 4, 2)
                    }
                    pib = "wrap"

                    def merge(u, v, g):
                        k = g // 2
                        ul = jnp.take(u, rot[k], mode=pib)
                        vr = jnp.take(v, rot[_LANES - k], mode=pib)
                        return jnp.where(masks[g], u + ul, v + vr)

                    brev = [0, 8, 4, 12, 2, 10, 6, 14,
                            1, 9, 5, 13, 3, 11, 7, 15]

                    @pl.loop(0, _CHUNK, step=_LANES)
                    def _(r0):
                        def edge_acc(j):
                            r = r0 + brev[j]
                            acc = None
                            for cc in range(nslc2):
                                sla = pl.ds(cc * _LANES, _LANES)
                                slb = pl.ds(half + cc * _LANES, _LANES)
                                av = plsc.bitcast(buf_a[b][r, sla], jnp.bfloat16)
                                bv = plsc.bitcast(buf_b[b][r, slb], jnp.bfloat16)
                                t = jnp.maximum(av + bv, 0.0)
                                p0, p1 = plsc.unpack(
                                    t * w2c[cc],
                                    format=plsc.PackFormat.INTERLEAVED)
                                ps = p0 + p1
                                acc = ps if acc is None else acc + ps
                            return acc

                        def build(lo, size):
                            if size == 2:
                                return merge(edge_acc(lo), edge_acc(lo + 1), 16)
                            half = size // 2
                            return merge(build(lo, half), build(lo + half, half),
                                         32 // size)

                        x = build(0, _LANES) + b2v
                        out_v[b][pl.ds(r0, _LANES)] = 1.0 / (1.0 + jnp.exp(-x))

                    base = (c * nw + wid) * _CHUNK
                    pltpu.async_copy(out_v[b], o_hbm.at[pl.ds(base, _CHUNK)],
                                     sem_o[b])

                    @pl.when(c + 2 < nv)
                    def _():
                        start_gathers(b2)

        for b in range(_NBUF):
            @pl.when(nv > b)
            def _():
                wait_out(b)

    return k(table_p, src, dst, w2bf, b2rep)


def kernel(latent_space, edge_index, W1, b1, W2, b2):
    d = latent_space.shape[1]
    src = edge_index[0].astype(jnp.int32)
    dst = edge_index[1].astype(jnp.int32)
    table_p = _precompute_table(latent_space, W1[:d], W1[d:], b1)
    w2bf = _pack_halves(W2[:, 0])
    b2rep = jnp.full((16,), b2[0], jnp.float32)
    return _sc_edge_decode(table_p, src, dst, w2bf, b2rep)


# speedup vs baseline: 2.4243x; 1.0212x over previous
import dataclasses
import functools

import jax
import jax.numpy as jnp
from jax import lax
from jax.experimental import pallas as pl
from jax.experimental.pallas import tpu as pltpu
from jax.experimental.pallas import tpu_sc as plsc

_SC_CORES = 2
_SC_SUBCORES = 16
_LANES = 16
_CHUNK = 128


def _pack_halves(v):
    k = v.shape[-1] // 2
    lo = v[..., :k].astype(jnp.bfloat16)
    hi = v[..., k:].astype(jnp.bfloat16)
    u1 = lax.bitcast_convert_type(lo, jnp.uint16).astype(jnp.uint32)
    u2 = lax.bitcast_convert_type(hi, jnp.uint16).astype(jnp.uint32)
    return lax.bitcast_convert_type(u1 | (u2 << 16), jnp.float32)


def _precompute_table(latent, w1a, w1b, b1):
    n, d = latent.shape
    h = w1a.shape[1]
    blk = 2000
    dn = (((1,), (0,)), ((), ()))

    def body(lat_ref, w1a_ref, w1b_ref, b1_ref, o_ref):
        x = lat_ref[...]
        a = lax.dot_general(x, w1a_ref[...], dn,
                            precision=lax.Precision.DEFAULT) + b1_ref[...]
        bt = lax.dot_general(x, w1b_ref[...], dn,
                             precision=lax.Precision.DEFAULT)
        o_ref[:, :h // 2] = _pack_halves(a)
        o_ref[:, h // 2:] = _pack_halves(bt)

    return pl.pallas_call(
        body,
        grid=(n // blk,),
        in_specs=[
            pl.BlockSpec((blk, d), lambda i: (i, 0)),
            pl.BlockSpec((d, h), lambda i: (0, 0)),
            pl.BlockSpec((d, h), lambda i: (0, 0)),
            pl.BlockSpec((1, h), lambda i: (0, 0)),
        ],
        out_specs=pl.BlockSpec((blk, h), lambda i: (i, 0)),
        out_shape=jax.ShapeDtypeStruct((n, h), jnp.float32),
    )(latent, w1a, w1b, b1.reshape(1, h))


_NBUF = 3


def _sc_edge_decode(table_p, src, dst, w2bf, b2rep):
    e = src.shape[0]
    h = table_p.shape[1]
    half = h // 2
    nslc2 = half // _LANES
    nw = _SC_CORES * _SC_SUBCORES
    n_chunks = e // _CHUNK
    per_worker = -(-n_chunks // nw)

    mesh = plsc.VectorSubcoreMesh(core_axis_name="c", subcore_axis_name="s")

    scratch = (
        [pltpu.VMEM((_CHUNK,), jnp.int32) for _ in range(2 * _NBUF)]
        + [pltpu.VMEM((_CHUNK, h), jnp.float32) for _ in range(2 * _NBUF)]
        + [pltpu.VMEM((_CHUNK,), jnp.float32) for _ in range(_NBUF)]
        + [pltpu.VMEM(w2bf.shape, jnp.float32),
           pltpu.VMEM(b2rep.shape, jnp.float32)]
        + [pltpu.SemaphoreType.DMA for _ in range(3 * _NBUF)]
    )

    cp = pltpu.CompilerParams()
    if "needs_layout_passes" in pltpu.CompilerParams.__dataclass_fields__:
        cp = dataclasses.replace(cp, needs_layout_passes=False)

    @functools.partial(
        pl.kernel,
        mesh=mesh,
        out_type=jax.ShapeDtypeStruct((e,), jnp.float32),
        scratch_types=scratch,
        compiler_params=cp,
    )
    def k(p_hbm, src_hbm, dst_hbm, w_hbm, b2_hbm, o_hbm, *bufs):
        idx_s = bufs[0:_NBUF]
        idx_d = bufs[_NBUF:2 * _NBUF]
        buf_a = bufs[2 * _NBUF:3 * _NBUF]
        buf_b = bufs[3 * _NBUF:4 * _NBUF]
        out_v = bufs[4 * _NBUF:5 * _NBUF]
        w_v = bufs[5 * _NBUF]
        b2_v = bufs[5 * _NBUF + 1]
        sem_g = bufs[5 * _NBUF + 2:6 * _NBUF + 2]
        sem_o = bufs[6 * _NBUF + 2:7 * _NBUF + 2]
        sem_i = bufs[7 * _NBUF + 2:8 * _NBUF + 2]

        wid = lax.axis_index("s") * _SC_CORES + lax.axis_index("c")
        nv = (n_chunks - 1 - wid) // nw + 1

        pltpu.sync_copy(w_hbm, w_v)
        pltpu.sync_copy(b2_hbm, b2_v)
        w2c = [plsc.bitcast(w_v[pl.ds(cc * _LANES, _LANES)], jnp.bfloat16)
               for cc in range(nslc2)]
        b2v = b2_v[pl.ds(0, _LANES)]

        def fetch_idx(c, b):
            base = (c * nw + wid) * _CHUNK
            pltpu.async_copy(src_hbm.at[pl.ds(base, _CHUNK)], idx_s[b],
                             sem_i[b])
            pltpu.async_copy(dst_hbm.at[pl.ds(base, _CHUNK)], idx_d[b],
                             sem_i[b])

        def start_gathers(b):
            pltpu.make_async_copy(src_hbm.at[pl.ds(0, _CHUNK)], idx_s[b],
                                  sem_i[b]).wait()
            pltpu.make_async_copy(dst_hbm.at[pl.ds(0, _CHUNK)], idx_d[b],
                                  sem_i[b]).wait()
            pltpu.async_copy(p_hbm.at[idx_s[b]], buf_a[b], sem_g[b])
            pltpu.async_copy(p_hbm.at[idx_d[b]], buf_b[b], sem_g[b])

        def wait_gathers(b):
            pltpu.make_async_copy(p_hbm.at[idx_s[b]], buf_a[b], sem_g[b]).wait()
            pltpu.make_async_copy(p_hbm.at[idx_d[b]], buf_b[b], sem_g[b]).wait()

        def wait_out(b):
            pltpu.make_async_copy(out_v[b], o_hbm.at[pl.ds(0, _CHUNK)],
                                  sem_o[b]).wait()

        for c0 in range(_NBUF):
            @pl.when(c0 < nv)
            def _():
                fetch_idx(c0, c0)
        for c0 in range(2):
            @pl.when(c0 < nv)
            def _():
                start_gathers(c0)

        @pl.loop(0, per_worker, step=_NBUF)
        def _(kk):
            for b in range(_NBUF):
                c = kk + b
                b2 = (b + 2) % _NBUF

                @pl.when(c < nv)
                def _():
                    @pl.when(c >= _NBUF)
                    def _():
                        wait_out(b)

                    wait_gathers(b)

                    @pl.when(c + _NBUF < nv)
                    def _():
                        fetch_idx(c + _NBUF, b)

                    lane = lax.iota(jnp.int32, _LANES)
                    rot = {
                        k: (lane + k) % _LANES
                        for k in (8, 4, 12, 2, 14, 1, 15)
                    }
                    masks = {
                        g: (lane % g) < (g // 2) for g in (16, 8, 4, 2)
                    }
                    pib = "wrap"

                    def merge(u, v, g):
                        k = g // 2
                        ul = jnp.take(u, rot[k], mode=pib)
                        vr = jnp.take(v, rot[_LANES - k], mode=pib)
                        return jnp.where(masks[g], u + ul, v + vr)

                    brev = [0, 8, 4, 12, 2, 10, 6, 14,
                            1, 9, 5, 13, 3, 11, 7, 15]

                    @plsc.parallel_loop(0, _CHUNK, step=_LANES, unroll=2)
                    def _(r0):
                        def edge_acc(j):
                            r = r0 + brev[j]
                            acc = None
                            for cc in range(nslc2):
                                sla = pl.ds(cc * _LANES, _LANES)
                                slb = pl.ds(half + cc * _LANES, _LANES)
                                av = plsc.bitcast(buf_a[b][r, sla], jnp.bfloat16)
                                bv = plsc.bitcast(buf_b[b][r, slb], jnp.bfloat16)
                                t = jnp.maximum(av + bv, 0.0)
                                p0, p1 = plsc.unpack(
                                    t * w2c[cc],
                                    format=plsc.PackFormat.INTERLEAVED)
                                ps = p0 + p1
                                acc = ps if acc is None else acc + ps
                            return acc

                        def build(lo, size):
                            if size == 2:
                                return merge(edge_acc(lo), edge_acc(lo + 1), 16)
                            half = size // 2
                            return merge(build(lo, half), build(lo + half, half),
                                         32 // size)

                        x = build(0, _LANES) + b2v
                        out_v[b][pl.ds(r0, _LANES)] = 1.0 / (1.0 + jnp.exp(-x))

                    base = (c * nw + wid) * _CHUNK
                    pltpu.async_copy(out_v[b], o_hbm.at[pl.ds(base, _CHUNK)],
                                     sem_o[b])

                    @pl.when(c + 2 < nv)
                    def _():
                        start_gathers(b2)

        for b in range(_NBUF):
            @pl.when(nv > b)
            def _():
                wait_out(b)

    return k(table_p, src, dst, w2bf, b2rep)


def kernel(latent_space, edge_index, W1, b1, W2, b2):
    d = latent_space.shape[1]
    src = edge_index[0].astype(jnp.int32)
    dst = edge_index[1].astype(jnp.int32)
    table_p = _precompute_table(latent_space, W1[:d], W1[d:], b1)
    w2bf = _pack_halves(W2[:, 0])
    b2rep = jnp.full((16,), b2[0], jnp.float32)
    return _sc_edge_decode(table_p, src, dst, w2bf, b2rep)
